# baseline (speedup 1.0000x reference)
import functools

import jax
import jax.numpy as jnp
from jax import lax
from jax.experimental import pallas as pl
from jax.experimental.pallas import tpu as pltpu
from jax.experimental.pallas import tpu_sc as plsc

N, D = 16384, 64

_info = plsc.get_sparse_core_info()
NC, NS, L = _info.num_cores, _info.num_subcores, _info.num_lanes
NW = NC * NS
ROWS = N // NW
CH = 4
CR = ROWS // CH
PW = L + 1

_mesh = plsc.VectorSubcoreMesh(core_axis_name="c", subcore_axis_name="s")


@functools.partial(
    pl.kernel,
    out_type=jax.ShapeDtypeStruct((N,), jnp.float32),
    mesh=_mesh,
    compiler_params=pltpu.CompilerParams(
        needs_layout_passes=False, use_tc_tiling_on_sc=False),
    scratch_types=[
        pltpu.VMEM((CR, D), jnp.float32),
        pltpu.VMEM((CR, D), jnp.float32),
        pltpu.VMEM((CR, D), jnp.float32),
        pltpu.VMEM((CR, D), jnp.float32),
        pltpu.VMEM((CR, PW), jnp.float32),
        pltpu.VMEM((ROWS,), jnp.float32),
        pltpu.SemaphoreType.DMA,
        pltpu.SemaphoreType.DMA,
    ],
)
def _rowdot(gu_hbm, gi_hbm, out_hbm, u0, i0, u1, i1, p_v, o_v, sem0, sem1):
    wid = lax.axis_index("s") * NC + lax.axis_index("c")
    base = wid * ROWS
    lanes = lax.iota(jnp.int32, L)
    bufs = ((u0, i0, sem0), (u1, i1, sem1))

    def start(c, buf):
        u, i, sem = buf
        cu = pltpu.async_copy(gu_hbm.at[pl.ds(base + c * CR, CR), :], u, sem)
        ci = pltpu.async_copy(gi_hbm.at[pl.ds(base + c * CR, CR), :], i, sem)
        return cu, ci

    pending = start(0, bufs[0])
    for c in range(CH):
        nxt = start(c + 1, bufs[(c + 1) % 2]) if c + 1 < CH else None
        pending[0].wait()
        pending[1].wait()
        pending = nxt
        u, i, _ = bufs[c % 2]

        def grp_body(g, carry, u=u, i=i, c=c):
            rb = g * L
            for l in range(L):
                r = rb + l
                acc = u[r, pl.ds(0, L)] * i[r, pl.ds(0, L)]
                for j in range(1, D // L):
                    acc = acc + u[r, pl.ds(j * L, L)] * i[r, pl.ds(j * L, L)]
                p_v[r, pl.ds(0, L)] = acc
            rows = rb + lanes
            cols = [plsc.load_gather(p_v, [rows, jnp.full((L,), j, jnp.int32)])
                    for j in range(L)]
            while len(cols) > 1:
                cols = [cols[k] + cols[k + 1] for k in range(0, len(cols), 2)]
            o_v[pl.ds(c * CR + g * L, L)] = cols[0]
            return carry

        lax.fori_loop(0, CR // L, grp_body, 0)

    pltpu.sync_copy(o_v, out_hbm.at[pl.ds(base, ROWS)])


def kernel(gu, gi):
    return _rowdot(gu, gi)


# speedup vs baseline: 1.8141x; 1.8141x over previous
import jax
import jax.numpy as jnp
from jax.experimental import pallas as pl
from jax.experimental.pallas import tpu as pltpu

N, D = 16384, 64
BR = 2048


def _body(u_ref, i_ref, o_ref):
    p = u_ref[...] * i_ref[...]
    o_ref[...] = jax.lax.dot_general(
        p, jnp.ones((D, 1), jnp.float32),
        (((1,), (0,)), ((), ())),
        preferred_element_type=jnp.float32)


def kernel(gu, gi):
    out = pl.pallas_call(
        _body,
        grid=(N // BR,),
        in_specs=[
            pl.BlockSpec((BR, D), lambda b: (b, 0)),
            pl.BlockSpec((BR, D), lambda b: (b, 0)),
        ],
        out_specs=pl.BlockSpec((BR, 1), lambda b: (b, 0)),
        out_shape=jax.ShapeDtypeStruct((N, 1), jnp.float32),
        compiler_params=pltpu.CompilerParams(
            dimension_semantics=("arbitrary",)),
    )(gu, gi)
    return jnp.squeeze(out, axis=1)


# speedup vs baseline: 7.8952x; 4.3521x over previous
import jax
import jax.numpy as jnp
from jax.experimental import pallas as pl
from jax.experimental.pallas import tpu as pltpu

N, D = 16384, 64
BC = 2048
NB = N // BC


def _body(u_hbm, i_hbm, o_hbm, ub0, ib0, ub1, ib1, o_v, sem0, sem1, osem):
    bufs = ((ub0, ib0, sem0), (ub1, ib1, sem1))

    def start(k, buf):
        u, i, sem = buf
        cu = pltpu.make_async_copy(u_hbm.at[:, pl.ds(k * BC, BC)], u, sem)
        ci = pltpu.make_async_copy(i_hbm.at[:, pl.ds(k * BC, BC)], i, sem)
        cu.start()
        ci.start()
        return cu, ci

    pend = start(0, bufs[0])
    for k in range(NB):
        nxt = start(k + 1, bufs[(k + 1) % 2]) if k + 1 < NB else None
        pend[0].wait()
        pend[1].wait()
        pend = nxt
        u, i, _ = bufs[k % 2]
        o_v[pl.ds(k * BC, BC)] = jnp.sum(u[...] * i[...], axis=0)
    out_cp = pltpu.make_async_copy(o_v, o_hbm, osem)
    out_cp.start()
    out_cp.wait()


def kernel(gu, gi):
    return pl.pallas_call(
        _body,
        in_specs=[
            pl.BlockSpec(memory_space=pltpu.HBM),
            pl.BlockSpec(memory_space=pltpu.HBM),
        ],
        out_specs=pl.BlockSpec(memory_space=pltpu.HBM),
        out_shape=jax.ShapeDtypeStruct((N,), jnp.float32),
        scratch_shapes=[
            pltpu.VMEM((D, BC), jnp.float32),
            pltpu.VMEM((D, BC), jnp.float32),
            pltpu.VMEM((D, BC), jnp.float32),
            pltpu.VMEM((D, BC), jnp.float32),
            pltpu.VMEM((N,), jnp.float32),
            pltpu.SemaphoreType.DMA,
            pltpu.SemaphoreType.DMA,
            pltpu.SemaphoreType.DMA,
        ],
    )(pltpu.with_memory_space_constraint(gu.T, pltpu.HBM),
      pltpu.with_memory_space_constraint(gi.T, pltpu.HBM))


# speedup vs baseline: 10.2072x; 1.2928x over previous
import jax
import jax.numpy as jnp
from jax.experimental import pallas as pl
from jax.experimental.pallas import tpu as pltpu

N, D = 16384, 64
BC = 4096
NB = N // BC
NBUF = 2


def _body(u_hbm, i_hbm, o_hbm, *rest):
    ubufs = rest[0:NBUF]
    ibufs = rest[NBUF:2 * NBUF]
    o_v = rest[2 * NBUF]
    sems = rest[2 * NBUF + 1:2 * NBUF + 1 + NBUF]
    osem = rest[2 * NBUF + 1 + NBUF]

    def start(k):
        b = k % NBUF
        cu = pltpu.make_async_copy(
            u_hbm.at[:, pl.ds(k * BC, BC)], ubufs[b], sems[b])
        ci = pltpu.make_async_copy(
            i_hbm.at[:, pl.ds(k * BC, BC)], ibufs[b], sems[b])
        cu.start()
        ci.start()
        return cu, ci

    pend = [start(k) for k in range(NBUF - 1)]
    for k in range(NB):
        if k + NBUF - 1 < NB:
            pend.append(start(k + NBUF - 1))
        cu, ci = pend.pop(0)
        cu.wait()
        ci.wait()
        b = k % NBUF
        o_v[pl.ds(k * BC, BC)] = jnp.sum(ubufs[b][...] * ibufs[b][...], axis=0)
    out_cp = pltpu.make_async_copy(o_v, o_hbm, osem)
    out_cp.start()
    out_cp.wait()


def kernel(gu, gi):
    return pl.pallas_call(
        _body,
        in_specs=[
            pl.BlockSpec(memory_space=pltpu.HBM),
            pl.BlockSpec(memory_space=pltpu.HBM),
        ],
        out_specs=pl.BlockSpec(memory_space=pltpu.HBM),
        out_shape=jax.ShapeDtypeStruct((N,), jnp.float32),
        scratch_shapes=(
            [pltpu.VMEM((D, BC), jnp.float32) for _ in range(2 * NBUF)]
            + [pltpu.VMEM((N,), jnp.float32)]
            + [pltpu.SemaphoreType.DMA for _ in range(NBUF + 1)]
        ),
    )(pltpu.with_memory_space_constraint(gu.T, pltpu.HBM),
      pltpu.with_memory_space_constraint(gi.T, pltpu.HBM))


# speedup vs baseline: 11.8123x; 1.1572x over previous
import jax
import jax.numpy as jnp
from jax.experimental import pallas as pl
from jax.experimental.pallas import tpu as pltpu

N, D = 16384, 64
BC = 2048
NB = N // BC
NBUF = 8


def _body(u_hbm, i_hbm, o_hbm, *rest):
    ubufs = rest[0:NBUF]
    ibufs = rest[NBUF:2 * NBUF]
    o_v = rest[2 * NBUF]
    sems = rest[2 * NBUF + 1:2 * NBUF + 1 + NBUF]
    osem = rest[2 * NBUF + 1 + NBUF]

    def start(k):
        b = k % NBUF
        cu = pltpu.make_async_copy(
            u_hbm.at[:, pl.ds(k * BC, BC)], ubufs[b], sems[b])
        ci = pltpu.make_async_copy(
            i_hbm.at[:, pl.ds(k * BC, BC)], ibufs[b], sems[b])
        cu.start()
        ci.start()
        return cu, ci

    pend = [start(k) for k in range(NBUF - 1)]
    for k in range(NB):
        if k + NBUF - 1 < NB:
            pend.append(start(k + NBUF - 1))
        cu, ci = pend.pop(0)
        cu.wait()
        ci.wait()
        b = k % NBUF
        o_v[pl.ds(k * BC, BC)] = jnp.sum(ubufs[b][...] * ibufs[b][...], axis=0)
    out_cp = pltpu.make_async_copy(o_v, o_hbm, osem)
    out_cp.start()
    out_cp.wait()


def kernel(gu, gi):
    return pl.pallas_call(
        _body,
        in_specs=[
            pl.BlockSpec(memory_space=pltpu.HBM),
            pl.BlockSpec(memory_space=pltpu.HBM),
        ],
        out_specs=pl.BlockSpec(memory_space=pltpu.HBM),
        out_shape=jax.ShapeDtypeStruct((N,), jnp.float32),
        scratch_shapes=(
            [pltpu.VMEM((D, BC), jnp.float32) for _ in range(2 * NBUF)]
            + [pltpu.VMEM((N,), jnp.float32)]
            + [pltpu.SemaphoreType.DMA for _ in range(NBUF + 1)]
        ),
    )(pltpu.with_memory_space_constraint(gu.T, pltpu.HBM),
      pltpu.with_memory_space_constraint(gi.T, pltpu.HBM))
